# baseline (device time: 670106 ns/iter reference)
import jax
import jax.numpy as jnp
from jax import lax
from jax.experimental import pallas as pl
from jax.experimental.pallas import tpu as pltpu

N_DEV = 8
N_CHUNKS_H = 4


def kernel(x, w_mat):
    m, k_sh = x.shape
    _, n = w_mat.shape
    mb = m // N_DEV
    nh = n // 2
    nc = nh // N_CHUNKS_H

    def body(x_ref, w_ref, out_ref, comm_f, comm_r, amax_buf,
             send_f, recv_f, send_r, recv_r,
             amax_send_sems, amax_recv_sems,
             credit_f, credit_r, out_sems):
        p = lax.axis_index("i")
        left = lax.rem(p - 1 + N_DEV, N_DEV)
        right = lax.rem(p + 1, N_DEV)

        barrier_sem = pltpu.get_barrier_semaphore()
        for nbr in (left, right):
            pl.semaphore_signal(
                barrier_sem, 1,
                device_id=(nbr,), device_id_type=pl.DeviceIdType.MESH,
            )
        pl.semaphore_wait(barrier_sem, 2)

        def split_x(b):
            xs = x_ref[pl.ds(b * mb, mb), :]
            xs_hi = xs.astype(jnp.bfloat16)
            xs_lo = (xs - xs_hi.astype(jnp.float32)).astype(jnp.bfloat16)
            return xs_hi, xs_lo

        def dot3(xsplit, col_lo):
            xs_hi, xs_lo = xsplit
            wc = w_ref[:, pl.ds(col_lo, nc)]
            w_hi = wc.astype(jnp.bfloat16)
            w_lo = (wc - w_hi.astype(jnp.float32)).astype(jnp.bfloat16)
            acc = jnp.dot(xs_hi, w_hi, preferred_element_type=jnp.float32)
            acc += jnp.dot(xs_hi, w_lo, preferred_element_type=jnp.float32)
            acc += jnp.dot(xs_lo, w_hi, preferred_element_type=jnp.float32)
            return acc

        def mk_send(comm, sems_s, sems_r, s, j, dev):
            cj = pl.ds(j * nc, nc)
            return pltpu.make_async_remote_copy(
                src_ref=comm.at[s % 2, :, cj],
                dst_ref=comm.at[(s + 1) % 2, :, cj],
                send_sem=sems_s.at[s % 2, j],
                recv_sem=sems_r.at[(s + 1) % 2, j],
                device_id=(dev,),
                device_id_type=pl.DeviceIdType.MESH,
            )

        xf = split_x(lax.rem(p - 1 + N_DEV, N_DEV))
        xr = split_x(lax.rem(p + 1, N_DEV))
        cur_f, cur_r = [], []
        for j in range(N_CHUNKS_H):
            cj = pl.ds(j * nc, nc)
            comm_f[0, :, cj] = dot3(xf, j * nc)
            rf = mk_send(comm_f, send_f, recv_f, 0, j, right)
            rf.start()
            cur_f.append(rf)
            comm_r[0, :, cj] = dot3(xr, nh + j * nc)
            rr = mk_send(comm_r, send_r, recv_r, 0, j, left)
            rr.start()
            cur_r.append(rr)

        amax_local = jnp.float32(0.0)

        for s in range(N_DEV - 1):
            ns = (s + 1) % 2
            last = s == N_DEV - 2
            xf = split_x(lax.rem(p - s - 2 + 2 * N_DEV, N_DEV))
            xr = split_x(lax.rem(p + s + 2, N_DEV))
            nxt_f, nxt_r = [], []
            for j in range(N_CHUNKS_H):
                cj = pl.ds(j * nc, nc)
                for (comm, cur, nxt, sems_s, sems_r, credit, up, down,
                     col0) in (
                    (comm_f, cur_f, nxt_f, send_f, recv_f, credit_f,
                     left, right, 0),
                    (comm_r, cur_r, nxt_r, send_r, recv_r, credit_r,
                     right, left, nh),
                ):
                    pb = dot3(xf if col0 == 0 else xr, col0 + j * nc)
                    rd = cur[j]
                    rd.wait_recv()
                    if last:
                        rd.wait_send()
                        yc = comm[1, :, cj] + pb
                        comm[0, :, cj] = yc
                        amax_local = jnp.maximum(
                            amax_local, jnp.max(jnp.abs(yc))
                        )
                    else:
                        comm[ns, :, cj] = comm[ns, :, cj] + pb
                        rd.wait_send()
                        pl.semaphore_signal(
                            credit.at[j], 1,
                            device_id=(up,),
                            device_id_type=pl.DeviceIdType.MESH,
                        )
                        pl.semaphore_wait(credit.at[j], 1)
                        nrd = mk_send(comm, sems_s, sems_r, s + 1, j, down)
                        nrd.start()
                        nxt.append(nrd)
            cur_f, cur_r = nxt_f, nxt_r

        amax_buf[pl.ds(0, 1), :] = jnp.full((1, 128), amax_local, jnp.float32)

        rds = []
        for d in range(1, N_DEV):
            tgt = lax.rem(p + d, N_DEV)
            rd = pltpu.make_async_remote_copy(
                src_ref=amax_buf.at[pl.ds(0, 1)],
                dst_ref=amax_buf.at[pl.ds(d, 1)],
                send_sem=amax_send_sems.at[d],
                recv_sem=amax_recv_sems.at[d],
                device_id=(tgt,),
                device_id_type=pl.DeviceIdType.MESH,
            )
            rd.start()
            rds.append(rd)
        for rd in rds:
            rd.wait_send()
        for rd in rds:
            rd.wait_recv()
        gmax = jnp.max(amax_buf[:, :])

        scale = gmax / 448.0
        inv_scale = 448.0 / gmax
        cps = []
        for comm, half in ((comm_f, 0), (comm_r, 1)):
            for j in range(N_CHUNKS_H):
                cj = pl.ds(j * nc, nc)
                q = jnp.clip(
                    comm[0, :, cj] * inv_scale, -448.0, 448.0
                ).astype(jnp.float8_e4m3fn)
                comm[0, :, cj] = q.astype(jnp.float32) * scale
                cp = pltpu.make_async_copy(
                    comm.at[0, :, cj],
                    out_ref.at[:, pl.ds(half * nh + j * nc, nc)],
                    out_sems.at[half * N_CHUNKS_H + j],
                )
                cp.start()
                cps.append(cp)
        for cp in cps:
            cp.wait()

    return pl.pallas_call(
        body,
        out_shape=jax.ShapeDtypeStruct((mb, n), jnp.float32),
        in_specs=[
            pl.BlockSpec(memory_space=pltpu.MemorySpace.VMEM),
            pl.BlockSpec(memory_space=pltpu.MemorySpace.VMEM),
        ],
        out_specs=pl.BlockSpec(memory_space=pltpu.MemorySpace.HBM),
        scratch_shapes=[
            pltpu.VMEM((2, mb, nh), jnp.float32),
            pltpu.VMEM((2, mb, nh), jnp.float32),
            pltpu.VMEM((N_DEV, 128), jnp.float32),
            pltpu.SemaphoreType.DMA((2, N_CHUNKS_H)),
            pltpu.SemaphoreType.DMA((2, N_CHUNKS_H)),
            pltpu.SemaphoreType.DMA((2, N_CHUNKS_H)),
            pltpu.SemaphoreType.DMA((2, N_CHUNKS_H)),
            pltpu.SemaphoreType.DMA((N_DEV,)),
            pltpu.SemaphoreType.DMA((N_DEV,)),
            pltpu.SemaphoreType.REGULAR((N_CHUNKS_H,)),
            pltpu.SemaphoreType.REGULAR((N_CHUNKS_H,)),
            pltpu.SemaphoreType.DMA((2 * N_CHUNKS_H,)),
        ],
        compiler_params=pltpu.CompilerParams(
            collective_id=0,
            vmem_limit_bytes=100 * 1024 * 1024,
        ),
    )(x, w_mat)
